# Initial kernel scaffold; baseline (speedup 1.0000x reference)
import jax
import jax.numpy as jnp
from jax.experimental import pallas as pl


def kernel(x, edge_index, mlp_W1, mlp_W2, mlp_bn_gamma, mlp_bn_beta, bn_gamma, bn_beta, pred_W, pred_b):
    raise NotImplementedError("write your pallas kernel here")


# speedup vs baseline: 3.6214x; 3.6214x over previous
import functools

import jax
import jax.numpy as jnp
from jax import lax
from jax.experimental import pallas as pl
from jax.experimental.pallas import tpu as pltpu
from jax.experimental.pallas import tpu_sc as plsc

N = 10000
E = 320000
D = 128
L = 4

N_PAD = 10240
ROWS_PER_TILE = N_PAD // 16
CHUNK = 128
NUM_WORKERS = 32
CHUNKS_PER_WORKER = 79
EDGES_PER_WORKER = CHUNKS_PER_WORKER * CHUNK
E_PAD = NUM_WORKERS * EDGES_PER_WORKER


def _seg_sum_body(h_hbm, srci_hbm, dsti_hbm, zeros_hbm, out_hbm,
                  idx_s, idx_d, rows_v, agg_sh, sem):
    c = lax.axis_index("c")
    s = lax.axis_index("s")
    wid = c * 16 + s
    base = wid * EDGES_PER_WORKER

    pltpu.sync_copy(zeros_hbm, rows_v)
    for k in range(ROWS_PER_TILE // CHUNK):
        pltpu.sync_copy(rows_v,
                        agg_sh.at[pl.ds(s * ROWS_PER_TILE + k * CHUNK, CHUNK)])
    plsc.subcore_barrier()

    def step(j, carry):
        pltpu.sync_copy(srci_hbm.at[pl.ds(base + j * CHUNK, CHUNK)], idx_s)
        pltpu.sync_copy(dsti_hbm.at[pl.ds(base + j * CHUNK, CHUNK)], idx_d)
        pltpu.async_copy(h_hbm.at[idx_s], rows_v, sem).wait()
        pltpu.sync_copy(rows_v, agg_sh.at[idx_d], add=True)
        return carry

    lax.fori_loop(0, CHUNKS_PER_WORKER, step, 0)
    plsc.subcore_barrier()

    for k in range(ROWS_PER_TILE // CHUNK):
        off = s * ROWS_PER_TILE + k * CHUNK
        pltpu.sync_copy(agg_sh.at[pl.ds(off, CHUNK)], rows_v)
        pltpu.sync_copy(rows_v, out_hbm.at[pl.ds(c * N_PAD + off, CHUNK)])


_seg_sum = functools.partial(
    pl.kernel,
    out_type=jax.ShapeDtypeStruct((2 * N_PAD, D), jnp.float32),
    mesh=plsc.VectorSubcoreMesh(core_axis_name="c", subcore_axis_name="s"),
    scratch_types=[
        pltpu.VMEM((CHUNK,), jnp.int32),
        pltpu.VMEM((CHUNK,), jnp.int32),
        pltpu.VMEM((CHUNK, D), jnp.float32),
        pltpu.VMEM_SHARED((N_PAD, D), jnp.float32),
        pltpu.SemaphoreType.DMA,
    ],
)(_seg_sum_body)


def _layer_body(h_ref, agg_ref, w1_ref, w2_ref, g1_ref, b1_ref, g2_ref,
                b2_ref, out_ref, pool_ref):
    agg = agg_ref[...]
    rst = h_ref[...] + agg[0, :N] + agg[1, :N]
    t = jnp.dot(rst, w1_ref[...], preferred_element_type=jnp.float32)
    mean = jnp.mean(t, axis=0)
    var = jnp.mean((t - mean) ** 2, axis=0)
    t = (t - mean) * lax.rsqrt(var + 1e-5) * g1_ref[...] + b1_ref[...]
    t = jnp.maximum(t, 0.0)
    h2 = jnp.dot(t, w2_ref[...], preferred_element_type=jnp.float32)
    mean2 = jnp.mean(h2, axis=0)
    var2 = jnp.mean((h2 - mean2) ** 2, axis=0)
    h2 = (h2 - mean2) * lax.rsqrt(var2 + 1e-5) * g2_ref[...] + b2_ref[...]
    h2 = jnp.maximum(h2, 0.0)
    out_ref[...] = h2
    pool_ref[...] = jnp.sum(h2, axis=0, keepdims=True)


def _tc_layer(h, agg, w1t, w2t, g1, b1, g2, b2):
    return pl.pallas_call(
        _layer_body,
        out_shape=(
            jax.ShapeDtypeStruct((N, D), jnp.float32),
            jax.ShapeDtypeStruct((1, D), jnp.float32),
        ),
    )(h, agg, w1t, w2t, g1, b1, g2, b2)


def _head_body(x_ref, pools_ref, wt_ref, b_ref, logp_ref, score_ref):
    score = jnp.sum(x_ref[...], axis=0, keepdims=True) @ wt_ref[0]
    score = score + b_ref[0:1, :]
    for i in range(L):
        score = score + pools_ref[i:i + 1, :] @ wt_ref[i + 1] + b_ref[i + 1:i + 2, :]
    m = jnp.max(score)
    lse = jnp.log(jnp.sum(jnp.exp(score - m))) + m
    logp_ref[...] = score - lse
    score_ref[...] = score


def _tc_head(x, pools, pred_wt, pred_b):
    return pl.pallas_call(
        _head_body,
        out_shape=(
            jax.ShapeDtypeStruct((1, D), jnp.float32),
            jax.ShapeDtypeStruct((1, D), jnp.float32),
        ),
    )(x, pools, pred_wt, pred_b)


def kernel(x, edge_index, mlp_W1, mlp_W2, mlp_bn_gamma, mlp_bn_beta,
           bn_gamma, bn_beta, pred_W, pred_b):
    n_fake = E_PAD - E
    src = jnp.concatenate([edge_index[0].astype(jnp.int32),
                           jnp.zeros((n_fake,), jnp.int32)])
    dst = jnp.concatenate([edge_index[1].astype(jnp.int32),
                           jnp.full((n_fake,), N, jnp.int32)])
    zeros = jnp.zeros((CHUNK, D), jnp.float32)
    w1t = mlp_W1.transpose(0, 2, 1)
    w2t = mlp_W2.transpose(0, 2, 1)
    pred_wt = pred_W.transpose(0, 2, 1)

    h = x
    pools = []
    for i in range(L):
        agg = _seg_sum(h, src, dst, zeros).reshape(2, N_PAD, D)
        h, pool = _tc_layer(h, agg, w1t[i], w2t[i], mlp_bn_gamma[i],
                            mlp_bn_beta[i], bn_gamma[i], bn_beta[i])
        pools.append(pool)
    pools = jnp.concatenate(pools, axis=0)
    logp, score = _tc_head(x, pools, pred_wt, pred_b)
    return (logp, score)
